# initial kernel scaffold (unmeasured)
import jax
import jax.numpy as jnp
from jax import lax
from jax.experimental import pallas as pl
from jax.experimental.pallas import tpu as pltpu

N_DEV = 8


def kernel(x, w_mat):
    m, _k_shard = x.shape
    n = w_mat.shape[1]
    chunk = m // N_DEV
    n_hops = N_DEV - 1

    def body(x_ref, w_ref, out_ref, stage_ref, rs_ref, ag_ref,
             send_sems, recv_sems):
        my = lax.axis_index("i")
        left = (my + N_DEV - 1) % N_DEV
        right = (my + 1) % N_DEV

        barrier = pltpu.get_barrier_semaphore()
        for nbr in (left, right):
            pl.semaphore_signal(
                barrier, inc=1,
                device_id=(nbr,), device_id_type=pl.DeviceIdType.MESH,
            )
        pl.semaphore_wait(barrier, 2)

        out_ref[:, :] = jnp.dot(
            x_ref[:, :], w_ref[:, :], preferred_element_type=jnp.float32
        )

        def rows(c):
            return pl.ds(c * chunk, chunk)

        for h in range(n_hops):
            s = (my + N_DEV - h) % N_DEV
            slot = h % 2
            stage_ref[slot, :, :] = out_ref[rows(s), :].astype(jnp.bfloat16)
            rdma = pltpu.make_async_remote_copy(
                src_ref=stage_ref.at[slot],
                dst_ref=rs_ref.at[h],
                send_sem=send_sems.at[h],
                recv_sem=recv_sems.at[h],
                device_id=(right,),
                device_id_type=pl.DeviceIdType.MESH,
            )
            rdma.start()
            rdma.wait()
            c = (my + N_DEV - 1 - h) % N_DEV
            out_ref[rows(c), :] = (
                out_ref[rows(c), :] + rs_ref[h].astype(jnp.float32)
            )

        mine = (my + 1) % N_DEV
        red = jnp.maximum(out_ref[rows(mine), :], 0.0)
        out_ref[rows(mine), :] = red
        stage_ref[0, :, :] = red.astype(jnp.bfloat16)

        for h in range(n_hops):
            src = stage_ref.at[0] if h == 0 else ag_ref.at[h - 1]
            rdma = pltpu.make_async_remote_copy(
                src_ref=src,
                dst_ref=ag_ref.at[h],
                send_sem=send_sems.at[n_hops + h],
                recv_sem=recv_sems.at[n_hops + h],
                device_id=(right,),
                device_id_type=pl.DeviceIdType.MESH,
            )
            rdma.start()
            rdma.wait()
            c = (my + N_DEV - h) % N_DEV
            out_ref[rows(c), :] = ag_ref[h].astype(jnp.float32)

    return pl.pallas_call(
        body,
        out_shape=jax.ShapeDtypeStruct((m, n), jnp.float32),
        in_specs=[
            pl.BlockSpec(memory_space=pltpu.VMEM),
            pl.BlockSpec(memory_space=pltpu.VMEM),
        ],
        out_specs=pl.BlockSpec(memory_space=pltpu.VMEM),
        scratch_shapes=[
            pltpu.VMEM((2, chunk, n), jnp.bfloat16),
            pltpu.VMEM((n_hops, chunk, n), jnp.bfloat16),
            pltpu.VMEM((n_hops, chunk, n), jnp.bfloat16),
            pltpu.SemaphoreType.DMA((2 * n_hops,)),
            pltpu.SemaphoreType.DMA((2 * n_hops,)),
        ],
        compiler_params=pltpu.CompilerParams(collective_id=0),
    )(x, w_mat)


# baseline (device time: 406656 ns/iter reference)
import jax
import jax.numpy as jnp
from jax import lax
from jax.experimental import pallas as pl
from jax.experimental.pallas import tpu as pltpu

N_DEV = 8


def kernel(x, w_mat):
    m, _k_shard = x.shape
    n = w_mat.shape[1]
    chunk = m // N_DEV
    n_hops = N_DEV - 1

    def body(x_ref, w_ref, out_ref, stage_ref, rs_ref, ag_ref,
             send_sems, recv_sems, credit_sem):
        my = lax.axis_index("i")
        left = (my + N_DEV - 1) % N_DEV
        right = (my + 1) % N_DEV

        barrier = pltpu.get_barrier_semaphore()
        for nbr in (left, right):
            pl.semaphore_signal(
                barrier, inc=1,
                device_id=(nbr,), device_id_type=pl.DeviceIdType.MESH,
            )
        pl.semaphore_wait(barrier, 2)

        def rows(c):
            return pl.ds(c * chunk, chunk)

        w_bf16 = w_ref[:, :].astype(jnp.bfloat16)
        for c in range(N_DEV):
            out_ref[rows(c), :] = jnp.dot(
                x_ref[rows(c), :].astype(jnp.bfloat16),
                w_bf16,
                preferred_element_type=jnp.float32,
            )

        for h in range(n_hops):
            s = (my + N_DEV - h) % N_DEV
            slot = h % 2
            stage_ref[slot, :, :] = out_ref[rows(s), :].astype(jnp.bfloat16)
            if h >= 2:
                pl.semaphore_wait(credit_sem, 1)
            rdma = pltpu.make_async_remote_copy(
                src_ref=stage_ref.at[slot],
                dst_ref=rs_ref.at[slot],
                send_sem=send_sems.at[h],
                recv_sem=recv_sems.at[h],
                device_id=(right,),
                device_id_type=pl.DeviceIdType.MESH,
            )
            rdma.start()
            rdma.wait()
            c = (my + N_DEV - 1 - h) % N_DEV
            out_ref[rows(c), :] = (
                out_ref[rows(c), :] + rs_ref[slot].astype(jnp.float32)
            )
            if h <= n_hops - 3:
                pl.semaphore_signal(
                    credit_sem, inc=1,
                    device_id=(left,), device_id_type=pl.DeviceIdType.MESH,
                )

        mine = (my + 1) % N_DEV
        red = jnp.maximum(out_ref[rows(mine), :], 0.0)
        out_ref[rows(mine), :] = red
        stage_ref[0, :, :] = red.astype(jnp.bfloat16)

        for h in range(n_hops):
            src = stage_ref.at[0] if h == 0 else ag_ref.at[(h - 1) % 2]
            if h >= 2:
                pl.semaphore_wait(credit_sem, 1)
            rdma = pltpu.make_async_remote_copy(
                src_ref=src,
                dst_ref=ag_ref.at[h % 2],
                send_sem=send_sems.at[n_hops + h],
                recv_sem=recv_sems.at[n_hops + h],
                device_id=(right,),
                device_id_type=pl.DeviceIdType.MESH,
            )
            rdma.start()
            rdma.wait()
            c = (my + N_DEV - h) % N_DEV
            out_ref[rows(c), :] = ag_ref[h % 2].astype(jnp.float32)
            if 1 <= h <= n_hops - 2:
                pl.semaphore_signal(
                    credit_sem, inc=1,
                    device_id=(left,), device_id_type=pl.DeviceIdType.MESH,
                )

    return pl.pallas_call(
        body,
        out_shape=jax.ShapeDtypeStruct((m, n), jnp.float32),
        in_specs=[
            pl.BlockSpec(memory_space=pltpu.VMEM),
            pl.BlockSpec(memory_space=pltpu.VMEM),
        ],
        out_specs=pl.BlockSpec(memory_space=pltpu.VMEM),
        scratch_shapes=[
            pltpu.VMEM((2, chunk, n), jnp.bfloat16),
            pltpu.VMEM((2, chunk, n), jnp.bfloat16),
            pltpu.VMEM((2, chunk, n), jnp.bfloat16),
            pltpu.SemaphoreType.DMA((2 * n_hops,)),
            pltpu.SemaphoreType.DMA((2 * n_hops,)),
            pltpu.SemaphoreType.REGULAR,
        ],
        compiler_params=pltpu.CompilerParams(
            collective_id=0,
            vmem_limit_bytes=64 * 1024 * 1024,
        ),
    )(x, w_mat)


# device time: 238097 ns/iter; 1.7079x vs baseline; 1.7079x over previous
import jax
import jax.numpy as jnp
from jax import lax
from jax.experimental import pallas as pl
from jax.experimental.pallas import tpu as pltpu

N_DEV = 8
SUB = 512
COL_PARTS = ((0, 768), (768, 640), (1408, 640))
DIM_ORDERS = (("x", "y", "z"), ("y", "z", "x"), ("z", "x", "y"))
MAX_SUBS = 4


def kernel(x, w_mat):
    m, _k_shard = x.shape
    n = w_mat.shape[1]
    chunk = m // N_DEV

    def body(x_ref, w_ref, out_ref, st0, st1, st2, rv0, rv1, rv2,
             send_sems, recv_sems, cr0, cr1, cr2):
        my = lax.axis_index("i")
        partners = {
            "x": (my ^ 1, (my ^ (my >> 1)) & 1),
            "y": (my ^ 3, (my >> 1) & 1),
            "z": (my ^ 4, (my >> 2) & 1),
        }
        stages = (st0, st1, st2)
        recvs = (rv0, rv1, rv2)
        credits = (cr0, cr1, cr2)

        barrier = pltpu.get_barrier_semaphore()
        for d in ("x", "y", "z"):
            pl.semaphore_signal(
                barrier, inc=1,
                device_id=(partners[d][0],),
                device_id_type=pl.DeviceIdType.MESH,
            )
        pl.semaphore_wait(barrier, 3)

        w_bf = w_ref[:, :].astype(jnp.bfloat16)
        for c in range(N_DEV):
            out_ref[pl.ds(c * chunk, chunk), :] = jnp.dot(
                x_ref[pl.ds(c * chunk, chunk), :].astype(jnp.bfloat16),
                w_bf,
                preferred_element_type=jnp.float32,
            )

        part_msg = [0, 0, 0]
        part_sent = ([], [], [])

        def send_msgs(p, row_start, n_rows, partner):
            c0, cw = COL_PARTS[p]
            descs = []
            for k in range(n_rows // SUB):
                mi = part_msg[p]
                part_msg[p] += 1
                slot = mi % 2
                if mi >= 2:
                    part_sent[p][mi - 2].wait_send()
                stages[p][slot, :, :] = out_ref[
                    pl.ds(row_start + k * SUB, SUB), pl.ds(c0, cw)
                ].astype(jnp.bfloat16)
                rdma = pltpu.make_async_remote_copy(
                    src_ref=stages[p].at[slot],
                    dst_ref=recvs[p].at[pl.ds(k * SUB, SUB)],
                    send_sem=send_sems.at[p * 2 + slot],
                    recv_sem=recv_sems.at[p * MAX_SUBS + k],
                    device_id=(partner,),
                    device_id_type=pl.DeviceIdType.MESH,
                )
                rdma.start()
                part_sent[p].append(rdma)
                descs.append(rdma)
            return descs

        def credit_to(p, partner):
            pl.semaphore_signal(
                credits[p], inc=1,
                device_id=(partner,), device_id_type=pl.DeviceIdType.MESH,
            )

        cur_start = [0, 0, 0]
        cur_len = [m, m, m]

        for r in range(3):
            pending = []
            for p in range(3):
                partner, b = partners[DIM_ORDERS[p][r]]
                half = cur_len[p] // 2
                send_start = cur_start[p] + (1 - b) * half
                keep_start = cur_start[p] + b * half
                if r > 0:
                    pl.semaphore_wait(credits[p], 1)
                descs = send_msgs(p, send_start, half, partner)
                pending.append((p, descs, keep_start))
                cur_start[p] = keep_start
                cur_len[p] = half
            for p, descs, keep_start in pending:
                c0, cw = COL_PARTS[p]
                for k, d in enumerate(descs):
                    d.wait_recv()
                    rows = pl.ds(keep_start + k * SUB, SUB)
                    out_ref[rows, pl.ds(c0, cw)] = (
                        out_ref[rows, pl.ds(c0, cw)]
                        + recvs[p][pl.ds(k * SUB, SUB), :].astype(jnp.float32)
                    )
                nxt_dim = DIM_ORDERS[p][r + 1] if r < 2 else DIM_ORDERS[p][2]
                credit_to(p, partners[nxt_dim][0])

        for p in range(3):
            c0, cw = COL_PARTS[p]
            rows = pl.ds(cur_start[p], cur_len[p])
            out_ref[rows, pl.ds(c0, cw)] = jnp.maximum(
                out_ref[rows, pl.ds(c0, cw)], 0.0
            )

        for r in range(3):
            pending = []
            for p in range(3):
                partner, b = partners[DIM_ORDERS[p][2 - r]]
                ln = cur_len[p]
                pl.semaphore_wait(credits[p], 1)
                descs = send_msgs(p, cur_start[p], ln, partner)
                partner_start = cur_start[p] + (1 - 2 * b) * ln
                pending.append((p, descs, partner_start))
                cur_start[p] = cur_start[p] - b * ln
                cur_len[p] = 2 * ln
            for p, descs, partner_start in pending:
                c0, cw = COL_PARTS[p]
                for k, d in enumerate(descs):
                    d.wait_recv()
                    out_ref[pl.ds(partner_start + k * SUB, SUB), pl.ds(c0, cw)] = (
                        recvs[p][pl.ds(k * SUB, SUB), :].astype(jnp.float32)
                    )
                if r < 2:
                    credit_to(p, partners[DIM_ORDERS[p][2 - r - 1]][0])

        for p in range(3):
            for d in part_sent[p][-2:]:
                d.wait_send()

    return pl.pallas_call(
        body,
        out_shape=jax.ShapeDtypeStruct((m, n), jnp.float32),
        in_specs=[
            pl.BlockSpec(memory_space=pltpu.VMEM),
            pl.BlockSpec(memory_space=pltpu.VMEM),
        ],
        out_specs=pl.BlockSpec(memory_space=pltpu.VMEM),
        scratch_shapes=[
            pltpu.VMEM((2, SUB, COL_PARTS[0][1]), jnp.bfloat16),
            pltpu.VMEM((2, SUB, COL_PARTS[1][1]), jnp.bfloat16),
            pltpu.VMEM((2, SUB, COL_PARTS[2][1]), jnp.bfloat16),
            pltpu.VMEM((m // 2, COL_PARTS[0][1]), jnp.bfloat16),
            pltpu.VMEM((m // 2, COL_PARTS[1][1]), jnp.bfloat16),
            pltpu.VMEM((m // 2, COL_PARTS[2][1]), jnp.bfloat16),
            pltpu.SemaphoreType.DMA((6,)),
            pltpu.SemaphoreType.DMA((3 * MAX_SUBS,)),
            pltpu.SemaphoreType.REGULAR,
            pltpu.SemaphoreType.REGULAR,
            pltpu.SemaphoreType.REGULAR,
        ],
        compiler_params=pltpu.CompilerParams(
            collective_id=0,
            vmem_limit_bytes=64 * 1024 * 1024,
        ),
    )(x, w_mat)
